# baseline (speedup 1.0000x reference)
import jax
import jax.numpy as jnp
from jax import lax
from jax.experimental import pallas as pl
from jax.experimental.pallas import tpu as pltpu
from jax.experimental.pallas import tpu_sc as plsc

N = 10000
E = 320000
D = 128

NC = 2
NS = 16
NW = NC * NS
EPW = E // NW
CHUNK = 40
TRIPS = EPW // CHUNK
G = 25
NG = TRIPS // G
ROWS_MAIN = 624
TAIL_START = ROWS_MAIN * NS
TAIL = N - TAIL_START

_mesh = plsc.VectorSubcoreMesh(core_axis_name="c", subcore_axis_name="s")


def _zero_1d(buf, n):
    for i in range(n // 16):
        buf[pl.ds(i * 16, 16)] = jnp.zeros((16,), jnp.float32)


def _deg_body(dst_hbm, out0, out1, acc, dst_grp, ones_v, zbuf, sem):
    cid = lax.axis_index("c")
    sid = lax.axis_index("s")
    wid = sid * NC + cid

    for off in range(0, CHUNK - 15, 16):
        ones_v[pl.ds(off, 16)] = jnp.full((16,), 1.0, jnp.float32)
    if CHUNK % 16:
        ones_v[pl.ds(CHUNK - 16, 16)] = jnp.full((16,), 1.0, jnp.float32)
    _zero_1d(zbuf, ROWS_MAIN + TAIL)

    start = sid * ROWS_MAIN
    pltpu.sync_copy(zbuf.at[pl.ds(0, ROWS_MAIN)], acc.at[pl.ds(start, ROWS_MAIN)])

    @pl.when(sid == NS - 1)
    def _():
        pltpu.sync_copy(zbuf.at[pl.ds(0, TAIL)], acc.at[pl.ds(TAIL_START, TAIL)])

    plsc.subcore_barrier()

    for g in range(NG):
        pltpu.sync_copy(dst_hbm.at[wid, g], dst_grp)
        handles = [
            pltpu.async_copy(ones_v, acc.at[dst_grp.at[i]], sem, add=True)
            for i in range(G)
        ]
        for h in handles:
            h.wait()

    plsc.subcore_barrier()

    def _readout(out):
        pltpu.sync_copy(acc.at[pl.ds(start, ROWS_MAIN)], zbuf.at[pl.ds(0, ROWS_MAIN)])
        pltpu.sync_copy(zbuf.at[pl.ds(0, ROWS_MAIN)], out.at[pl.ds(start, ROWS_MAIN)])

        @pl.when(sid == NS - 1)
        def _():
            pltpu.sync_copy(acc.at[pl.ds(TAIL_START, TAIL)], zbuf.at[pl.ds(0, TAIL)])
            pltpu.sync_copy(zbuf.at[pl.ds(0, TAIL)], out.at[pl.ds(TAIL_START, TAIL)])

    @pl.when(cid == 0)
    def _():
        _readout(out0)

    @pl.when(cid == 1)
    def _():
        _readout(out1)


_deg_call = pl.kernel(
    _deg_body,
    out_type=[
        jax.ShapeDtypeStruct((N,), jnp.float32),
        jax.ShapeDtypeStruct((N,), jnp.float32),
    ],
    mesh=_mesh,
    scratch_types=[
        pltpu.VMEM_SHARED((N,), jnp.float32),
        pltpu.VMEM((G, CHUNK), jnp.int32),
        pltpu.VMEM((CHUNK,), jnp.float32),
        pltpu.VMEM((ROWS_MAIN + TAIL,), jnp.float32),
        pltpu.SemaphoreType.DMA,
    ],
)

_RC = 40
_ROW_CHUNKS = [(j * _RC, _RC) for j in range(ROWS_MAIN // _RC)]
_ROW_CHUNKS.append((ROWS_MAIN - ROWS_MAIN % _RC, ROWS_MAIN % _RC))


def _agg_body(m_hbm, src_hbm, dst_hbm, out0, out1,
              acc, src_grp, dst_grp, buf_a, buf_b, buf_c, buf_d,
              gsem_a, gsem_b, gsem_c, gsem_d, ssem_a, ssem_b, ssem_c, ssem_d,
              sem_i):
    cid = lax.axis_index("c")
    sid = lax.axis_index("s")
    wid = sid * NC + cid

    def zrow(r, carry):
        for c in range(D // 16):
            buf_a[r, pl.ds(c * 16, 16)] = jnp.zeros((16,), jnp.float32)
        return carry

    lax.fori_loop(0, CHUNK, zrow, 0)

    start = sid * ROWS_MAIN
    zhandles = [
        pltpu.async_copy(buf_a.at[pl.ds(0, sz)], acc.at[pl.ds(start + off, sz)], sem_i)
        for off, sz in _ROW_CHUNKS
    ]

    @pl.when(sid == NS - 1)
    def _():
        pltpu.sync_copy(buf_a.at[pl.ds(0, TAIL)], acc.at[pl.ds(TAIL_START, TAIL)])

    for h in zhandles:
        h.wait()

    pltpu.sync_copy(src_hbm.at[wid, 0], src_grp.at[0])
    pltpu.sync_copy(dst_hbm.at[wid, 0], dst_grp.at[0])
    pltpu.async_copy(src_hbm.at[wid, 1], src_grp.at[1], sem_i)
    pltpu.async_copy(dst_hbm.at[wid, 1], dst_grp.at[1], sem_i)
    plsc.subcore_barrier()

    bufs = (buf_a, buf_b, buf_c, buf_d)
    gsems = (gsem_a, gsem_b, gsem_c, gsem_d)
    ssems = (ssem_a, ssem_b, ssem_c, ssem_d)

    def gather(c):
        b = c % 4
        pltpu.async_copy(m_hbm.at[src_grp.at[(c // G) % 2, c % G]], bufs[b], gsems[b])

    def gather_wait(c):
        b = c % 4
        pltpu.make_async_copy(m_hbm.at[src_grp.at[(c // G) % 2, c % G]], bufs[b], gsems[b]).wait()

    def scat(c):
        b = c % 4
        pltpu.async_copy(bufs[b], acc.at[dst_grp.at[(c // G) % 2, c % G]], ssems[b], add=True)

    def scat_wait(c):
        b = c % 4
        pltpu.make_async_copy(bufs[b], acc.at[dst_grp.at[(c // G) % 2, c % G]], ssems[b]).wait()

    def idx_prefetch(g):
        pltpu.async_copy(src_hbm.at[wid, g], src_grp.at[g % 2], sem_i)
        pltpu.async_copy(dst_hbm.at[wid, g], dst_grp.at[g % 2], sem_i)

    def idx_drain(g):
        pltpu.make_async_copy(src_hbm.at[wid, g], src_grp.at[g % 2], sem_i).wait()
        pltpu.make_async_copy(dst_hbm.at[wid, g], dst_grp.at[g % 2], sem_i).wait()

    gather(0)
    gather(1)
    for c in range(TRIPS):
        g, i = divmod(c, G)
        if i == 0 and g > 0 and g + 1 < NG:
            idx_prefetch(g + 1)
        if i == G - 2 and g + 1 < NG:
            idx_drain(g + 1)
        if c + 2 < TRIPS:
            if c >= 2:
                scat_wait(c - 2)
            gather(c + 2)
        gather_wait(c)
        scat(c)
    scat_wait(TRIPS - 4)
    scat_wait(TRIPS - 3)
    scat_wait(TRIPS - 2)
    scat_wait(TRIPS - 1)

    plsc.subcore_barrier()

    def _readout(out):
        nk = len(_ROW_CHUNKS)

        def rin(k):
            off, sz = _ROW_CHUNKS[k]
            return pltpu.make_async_copy(
                acc.at[pl.ds(start + off, sz)], bufs[k % 4].at[pl.ds(0, sz)], gsems[k % 4])

        def rout(k):
            off, sz = _ROW_CHUNKS[k]
            return pltpu.make_async_copy(
                bufs[k % 4].at[pl.ds(0, sz)], out.at[pl.ds(start + off, sz)], ssems[k % 4])

        for k in range(nk + 1):
            if k < nk:
                if k >= 4:
                    rout(k - 4).wait()
                rin(k).start()
            if k >= 1:
                rin(k - 1).wait()
                rout(k - 1).start()
        for k in range(max(0, nk - 4), nk):
            rout(k).wait()

        @pl.when(sid == NS - 1)
        def _():
            pltpu.sync_copy(acc.at[pl.ds(TAIL_START, TAIL)], buf_a.at[pl.ds(0, TAIL)])
            pltpu.sync_copy(buf_a.at[pl.ds(0, TAIL)], out.at[pl.ds(TAIL_START, TAIL)])

    @pl.when(cid == 0)
    def _():
        _readout(out0)

    @pl.when(cid == 1)
    def _():
        _readout(out1)


_agg_call = pl.kernel(
    _agg_body,
    out_type=[
        jax.ShapeDtypeStruct((N, D), jnp.float32),
        jax.ShapeDtypeStruct((N, D), jnp.float32),
    ],
    mesh=_mesh,
    scratch_types=[
        pltpu.VMEM_SHARED((N, D), jnp.float32),
        pltpu.VMEM((2, G, CHUNK), jnp.int32),
        pltpu.VMEM((2, G, CHUNK), jnp.int32),
        pltpu.VMEM((CHUNK, D), jnp.float32),
        pltpu.VMEM((CHUNK, D), jnp.float32),
        pltpu.VMEM((CHUNK, D), jnp.float32),
        pltpu.VMEM((CHUNK, D), jnp.float32),
        pltpu.SemaphoreType.DMA,
        pltpu.SemaphoreType.DMA,
        pltpu.SemaphoreType.DMA,
        pltpu.SemaphoreType.DMA,
        pltpu.SemaphoreType.DMA,
        pltpu.SemaphoreType.DMA,
        pltpu.SemaphoreType.DMA,
        pltpu.SemaphoreType.DMA,
        pltpu.SemaphoreType.DMA,
    ],
)


BN = 2000
GRID = N // BN

_INV_SQRT2 = 0.7071067811865476


def _dinv_block(d0, d1):
    return jnp.broadcast_to(lax.rsqrt(d0 + d1 + 1.0), (BN, D))


def _gelu(t):
    return 0.5 * t * (1.0 + lax.erf(t * _INV_SQRT2))


def _tcA_body(x_ref, win_ref, bin_ref, w0_ref, h0_ref, hw_ref):
    h0 = jnp.dot(x_ref[...], win_ref[...], preferred_element_type=jnp.float32)
    h0 = h0 + bin_ref[...]
    hw = jnp.dot(h0, w0_ref[...], preferred_element_type=jnp.float32)
    h0_ref[...] = h0
    hw_ref[...] = hw


def _tcM_body(hw_ref, d0_ref, d1_ref, m0_ref):
    dinv = _dinv_block(d0_ref[...], d1_ref[...])
    m0_ref[...] = hw_ref[...] * dinv


def _tcB_body(h_ref, m_ref, s0_ref, s1_ref, d0_ref, d1_ref, b_ref, wn_ref,
              hn_ref, mn_ref):
    dinv = _dinv_block(d0_ref[...], d1_ref[...])
    t = dinv * (s0_ref[...] + s1_ref[...] + m_ref[...]) + b_ref[...]
    hn = _gelu(t) + h_ref[...]
    hw = jnp.dot(hn, wn_ref[...], preferred_element_type=jnp.float32)
    hn_ref[...] = hn
    mn_ref[...] = hw * dinv


def _tcC_body(h_ref, m_ref, s0_ref, s1_ref, d0_ref, d1_ref, b_ref, hn_ref):
    dinv = _dinv_block(d0_ref[...], d1_ref[...])
    t = dinv * (s0_ref[...] + s1_ref[...] + m_ref[...]) + b_ref[...]
    hn_ref[...] = _gelu(t) + h_ref[...]


_row_spec = pl.BlockSpec((BN, D), lambda i: (i, 0))
_w_spec = pl.BlockSpec((D, D), lambda i: (0, 0))
_b_spec = pl.BlockSpec((1, D), lambda i: (0, 0))
_d_spec = pl.BlockSpec((BN, 1), lambda i: (i, 0))

_tcA_call = pl.pallas_call(
    _tcA_body,
    grid=(GRID,),
    in_specs=[_row_spec, _w_spec, _b_spec, _w_spec],
    out_specs=[_row_spec, _row_spec],
    out_shape=[jax.ShapeDtypeStruct((N, D), jnp.float32)] * 2,
)

_tcM_call = pl.pallas_call(
    _tcM_body,
    grid=(GRID,),
    in_specs=[_row_spec, _d_spec, _d_spec],
    out_specs=pl.BlockSpec((BN, D), lambda i: (i, 0)),
    out_shape=jax.ShapeDtypeStruct((N, D), jnp.float32),
)

_tcB_call = pl.pallas_call(
    _tcB_body,
    grid=(GRID,),
    in_specs=[_row_spec, _row_spec, _row_spec, _row_spec, _d_spec, _d_spec,
              _b_spec, _w_spec],
    out_specs=[_row_spec, _row_spec],
    out_shape=[jax.ShapeDtypeStruct((N, D), jnp.float32)] * 2,
)

_tcC_call = pl.pallas_call(
    _tcC_body,
    grid=(GRID,),
    in_specs=[_row_spec, _row_spec, _row_spec, _row_spec, _d_spec, _d_spec,
              _b_spec],
    out_specs=pl.BlockSpec((BN, D), lambda i: (i, 0)),
    out_shape=jax.ShapeDtypeStruct((N, D), jnp.float32),
)


def kernel(x, edge_index, W_in, b_in, W0, b0, W1, b1, W2, b2):
    src = edge_index[0].reshape(NW, NG, G, CHUNK)
    dst = edge_index[1].reshape(NW, NG, G, CHUNK)

    d0, d1 = _deg_call(dst)
    d0 = d0.reshape(N, 1)
    d1 = d1.reshape(N, 1)

    h0, hw0 = _tcA_call(x, W_in, b_in.reshape(1, D), W0)
    m0 = _tcM_call(hw0, d0, d1)
    s0a, s0b = _agg_call(m0, src, dst)
    h1, m1 = _tcB_call(h0, m0, s0a, s0b, d0, d1, b0.reshape(1, D), W1)
    s1a, s1b = _agg_call(m1, src, dst)
    h2, m2 = _tcB_call(h1, m1, s1a, s1b, d0, d1, b1.reshape(1, D), W2)
    s2a, s2b = _agg_call(m2, src, dst)
    h3 = _tcC_call(h2, m2, s2a, s2b, d0, d1, b2.reshape(1, D))
    return h3


# speedup vs baseline: 1.1120x; 1.1120x over previous
import jax
import jax.numpy as jnp
from jax import lax
from jax.experimental import pallas as pl
from jax.experimental.pallas import tpu as pltpu
from jax.experimental.pallas import tpu_sc as plsc

N = 10000
E = 320000
D = 128

NC = 2
NS = 16
NW = NC * NS
EPW = E // NW
CHUNK = 80
TRIPS = EPW // CHUNK
G = 25
NG = TRIPS // G
ROWS_MAIN = 624
TAIL_START = ROWS_MAIN * NS
TAIL = N - TAIL_START

_mesh = plsc.VectorSubcoreMesh(core_axis_name="c", subcore_axis_name="s")


def _zero_1d(buf, n):
    for i in range(n // 16):
        buf[pl.ds(i * 16, 16)] = jnp.zeros((16,), jnp.float32)


def _deg_body(dst_hbm, out0, out1, acc, dst_grp, ones_v, zbuf, sem):
    cid = lax.axis_index("c")
    sid = lax.axis_index("s")
    wid = sid * NC + cid

    for i in range(CHUNK // 16):
        ones_v[pl.ds(i * 16, 16)] = jnp.full((16,), 1.0, jnp.float32)
    _zero_1d(zbuf, ROWS_MAIN + TAIL)

    start = sid * ROWS_MAIN
    pltpu.sync_copy(zbuf.at[pl.ds(0, ROWS_MAIN)], acc.at[pl.ds(start, ROWS_MAIN)])

    @pl.when(sid == NS - 1)
    def _():
        pltpu.sync_copy(zbuf.at[pl.ds(0, TAIL)], acc.at[pl.ds(TAIL_START, TAIL)])

    plsc.subcore_barrier()

    for g in range(NG):
        pltpu.sync_copy(dst_hbm.at[wid, g], dst_grp)
        handles = [
            pltpu.async_copy(ones_v, acc.at[dst_grp.at[i]], sem, add=True)
            for i in range(G)
        ]
        for h in handles:
            h.wait()

    plsc.subcore_barrier()

    def _readout(out):
        pltpu.sync_copy(acc.at[pl.ds(start, ROWS_MAIN)], zbuf.at[pl.ds(0, ROWS_MAIN)])
        pltpu.sync_copy(zbuf.at[pl.ds(0, ROWS_MAIN)], out.at[pl.ds(start, ROWS_MAIN)])

        @pl.when(sid == NS - 1)
        def _():
            pltpu.sync_copy(acc.at[pl.ds(TAIL_START, TAIL)], zbuf.at[pl.ds(0, TAIL)])
            pltpu.sync_copy(zbuf.at[pl.ds(0, TAIL)], out.at[pl.ds(TAIL_START, TAIL)])

    @pl.when(cid == 0)
    def _():
        _readout(out0)

    @pl.when(cid == 1)
    def _():
        _readout(out1)


_deg_call = pl.kernel(
    _deg_body,
    out_type=[
        jax.ShapeDtypeStruct((N,), jnp.float32),
        jax.ShapeDtypeStruct((N,), jnp.float32),
    ],
    mesh=_mesh,
    scratch_types=[
        pltpu.VMEM_SHARED((N,), jnp.float32),
        pltpu.VMEM((G, CHUNK), jnp.int32),
        pltpu.VMEM((CHUNK,), jnp.float32),
        pltpu.VMEM((ROWS_MAIN + TAIL,), jnp.float32),
        pltpu.SemaphoreType.DMA,
    ],
)

_RC = 80
_ROW_CHUNKS = [(j * _RC, _RC) for j in range(ROWS_MAIN // _RC)]
_ROW_CHUNKS.append((ROWS_MAIN - ROWS_MAIN % _RC, ROWS_MAIN % _RC))


def _agg_body(m_hbm, src_hbm, dst_hbm, out0, out1,
              acc, src_grp, dst_grp, buf_a, buf_b, buf_c,
              gsem_a, gsem_b, gsem_c, ssem_a, ssem_b, ssem_c, sem_i):
    cid = lax.axis_index("c")
    sid = lax.axis_index("s")
    wid = sid * NC + cid

    def zrow(r, carry):
        for c in range(D // 16):
            buf_a[r, pl.ds(c * 16, 16)] = jnp.zeros((16,), jnp.float32)
        return carry

    lax.fori_loop(0, CHUNK, zrow, 0)

    start = sid * ROWS_MAIN
    zhandles = [
        pltpu.async_copy(buf_a.at[pl.ds(0, sz)], acc.at[pl.ds(start + off, sz)], sem_i)
        for off, sz in _ROW_CHUNKS
    ]

    @pl.when(sid == NS - 1)
    def _():
        pltpu.sync_copy(buf_a.at[pl.ds(0, TAIL)], acc.at[pl.ds(TAIL_START, TAIL)])

    for h in zhandles:
        h.wait()

    pltpu.sync_copy(src_hbm.at[wid, 0], src_grp.at[0])
    pltpu.sync_copy(dst_hbm.at[wid, 0], dst_grp.at[0])
    pltpu.async_copy(src_hbm.at[wid, 1], src_grp.at[1], sem_i)
    pltpu.async_copy(dst_hbm.at[wid, 1], dst_grp.at[1], sem_i)
    plsc.subcore_barrier()

    bufs = (buf_a, buf_b, buf_c)
    gsems = (gsem_a, gsem_b, gsem_c)
    ssems = (ssem_a, ssem_b, ssem_c)

    def gather(c):
        b = c % 3
        pltpu.async_copy(m_hbm.at[src_grp.at[(c // G) % 2, c % G]], bufs[b], gsems[b])

    def gather_wait(c):
        b = c % 3
        pltpu.make_async_copy(m_hbm.at[src_grp.at[(c // G) % 2, c % G]], bufs[b], gsems[b]).wait()

    def scat(c):
        b = c % 3
        pltpu.async_copy(bufs[b], acc.at[dst_grp.at[(c // G) % 2, c % G]], ssems[b], add=True)

    def scat_wait(c):
        b = c % 3
        pltpu.make_async_copy(bufs[b], acc.at[dst_grp.at[(c // G) % 2, c % G]], ssems[b]).wait()

    def idx_prefetch(g):
        pltpu.async_copy(src_hbm.at[wid, g], src_grp.at[g % 2], sem_i)
        pltpu.async_copy(dst_hbm.at[wid, g], dst_grp.at[g % 2], sem_i)

    def idx_drain(g):
        pltpu.make_async_copy(src_hbm.at[wid, g], src_grp.at[g % 2], sem_i).wait()
        pltpu.make_async_copy(dst_hbm.at[wid, g], dst_grp.at[g % 2], sem_i).wait()

    gather(0)
    gather(1)
    for c in range(TRIPS):
        g, i = divmod(c, G)
        if i == 0 and g > 0 and g + 1 < NG:
            idx_prefetch(g + 1)
        if i == G - 2 and g + 1 < NG:
            idx_drain(g + 1)
        if c + 2 < TRIPS:
            if c >= 1:
                scat_wait(c - 1)
            gather(c + 2)
        gather_wait(c)
        scat(c)
    scat_wait(TRIPS - 3)
    scat_wait(TRIPS - 2)
    scat_wait(TRIPS - 1)

    plsc.subcore_barrier()

    def _readout(out):
        nk = len(_ROW_CHUNKS)

        def rin(k):
            off, sz = _ROW_CHUNKS[k]
            return pltpu.make_async_copy(
                acc.at[pl.ds(start + off, sz)], bufs[k % 3].at[pl.ds(0, sz)], gsems[k % 3])

        def rout(k):
            off, sz = _ROW_CHUNKS[k]
            return pltpu.make_async_copy(
                bufs[k % 3].at[pl.ds(0, sz)], out.at[pl.ds(start + off, sz)], ssems[k % 3])

        for k in range(nk + 1):
            if k < nk:
                if k >= 3:
                    rout(k - 3).wait()
                rin(k).start()
            if k >= 1:
                rin(k - 1).wait()
                rout(k - 1).start()
        for k in range(max(0, nk - 3), nk):
            rout(k).wait()

        @pl.when(sid == NS - 1)
        def _():
            pltpu.sync_copy(acc.at[pl.ds(TAIL_START, TAIL)], buf_a.at[pl.ds(0, TAIL)])
            pltpu.sync_copy(buf_a.at[pl.ds(0, TAIL)], out.at[pl.ds(TAIL_START, TAIL)])

    @pl.when(cid == 0)
    def _():
        _readout(out0)

    @pl.when(cid == 1)
    def _():
        _readout(out1)


_agg_call = pl.kernel(
    _agg_body,
    out_type=[
        jax.ShapeDtypeStruct((N, D), jnp.float32),
        jax.ShapeDtypeStruct((N, D), jnp.float32),
    ],
    mesh=_mesh,
    scratch_types=[
        pltpu.VMEM_SHARED((N, D), jnp.float32),
        pltpu.VMEM((2, G, CHUNK), jnp.int32),
        pltpu.VMEM((2, G, CHUNK), jnp.int32),
        pltpu.VMEM((CHUNK, D), jnp.float32),
        pltpu.VMEM((CHUNK, D), jnp.float32),
        pltpu.VMEM((CHUNK, D), jnp.float32),
        pltpu.SemaphoreType.DMA,
        pltpu.SemaphoreType.DMA,
        pltpu.SemaphoreType.DMA,
        pltpu.SemaphoreType.DMA,
        pltpu.SemaphoreType.DMA,
        pltpu.SemaphoreType.DMA,
        pltpu.SemaphoreType.DMA,
    ],
)


BN = 2000
GRID = N // BN

_INV_SQRT2 = 0.7071067811865476


def _dinv_block(d0, d1):
    return jnp.broadcast_to(lax.rsqrt(d0 + d1 + 1.0), (BN, D))


def _gelu(t):
    return 0.5 * t * (1.0 + lax.erf(t * _INV_SQRT2))


def _tcA_body(x_ref, win_ref, bin_ref, w0_ref, h0_ref, hw_ref):
    h0 = jnp.dot(x_ref[...], win_ref[...], preferred_element_type=jnp.float32)
    h0 = h0 + bin_ref[...]
    hw = jnp.dot(h0, w0_ref[...], preferred_element_type=jnp.float32)
    h0_ref[...] = h0
    hw_ref[...] = hw


def _tcM_body(hw_ref, d0_ref, d1_ref, m0_ref):
    dinv = _dinv_block(d0_ref[...], d1_ref[...])
    m0_ref[...] = hw_ref[...] * dinv


def _tcB_body(h_ref, m_ref, s0_ref, s1_ref, d0_ref, d1_ref, b_ref, wn_ref,
              hn_ref, mn_ref):
    dinv = _dinv_block(d0_ref[...], d1_ref[...])
    t = dinv * (s0_ref[...] + s1_ref[...] + m_ref[...]) + b_ref[...]
    hn = _gelu(t) + h_ref[...]
    hw = jnp.dot(hn, wn_ref[...], preferred_element_type=jnp.float32)
    hn_ref[...] = hn
    mn_ref[...] = hw * dinv


def _tcC_body(h_ref, m_ref, s0_ref, s1_ref, d0_ref, d1_ref, b_ref, hn_ref):
    dinv = _dinv_block(d0_ref[...], d1_ref[...])
    t = dinv * (s0_ref[...] + s1_ref[...] + m_ref[...]) + b_ref[...]
    hn_ref[...] = _gelu(t) + h_ref[...]


_row_spec = pl.BlockSpec((BN, D), lambda i: (i, 0))
_w_spec = pl.BlockSpec((D, D), lambda i: (0, 0))
_b_spec = pl.BlockSpec((1, D), lambda i: (0, 0))
_d_spec = pl.BlockSpec((BN, 1), lambda i: (i, 0))

_tcA_call = pl.pallas_call(
    _tcA_body,
    grid=(GRID,),
    in_specs=[_row_spec, _w_spec, _b_spec, _w_spec],
    out_specs=[_row_spec, _row_spec],
    out_shape=[jax.ShapeDtypeStruct((N, D), jnp.float32)] * 2,
)

_tcM_call = pl.pallas_call(
    _tcM_body,
    grid=(GRID,),
    in_specs=[_row_spec, _d_spec, _d_spec],
    out_specs=pl.BlockSpec((BN, D), lambda i: (i, 0)),
    out_shape=jax.ShapeDtypeStruct((N, D), jnp.float32),
)

_tcB_call = pl.pallas_call(
    _tcB_body,
    grid=(GRID,),
    in_specs=[_row_spec, _row_spec, _row_spec, _row_spec, _d_spec, _d_spec,
              _b_spec, _w_spec],
    out_specs=[_row_spec, _row_spec],
    out_shape=[jax.ShapeDtypeStruct((N, D), jnp.float32)] * 2,
)

_tcC_call = pl.pallas_call(
    _tcC_body,
    grid=(GRID,),
    in_specs=[_row_spec, _row_spec, _row_spec, _row_spec, _d_spec, _d_spec,
              _b_spec],
    out_specs=pl.BlockSpec((BN, D), lambda i: (i, 0)),
    out_shape=jax.ShapeDtypeStruct((N, D), jnp.float32),
)


def kernel(x, edge_index, W_in, b_in, W0, b0, W1, b1, W2, b2):
    src = edge_index[0].reshape(NW, NG, G, CHUNK)
    dst = edge_index[1].reshape(NW, NG, G, CHUNK)

    d0, d1 = _deg_call(dst)
    d0 = d0.reshape(N, 1)
    d1 = d1.reshape(N, 1)

    h0, hw0 = _tcA_call(x, W_in, b_in.reshape(1, D), W0)
    m0 = _tcM_call(hw0, d0, d1)
    s0a, s0b = _agg_call(m0, src, dst)
    h1, m1 = _tcB_call(h0, m0, s0a, s0b, d0, d1, b0.reshape(1, D), W1)
    s1a, s1b = _agg_call(m1, src, dst)
    h2, m2 = _tcB_call(h1, m1, s1a, s1b, d0, d1, b1.reshape(1, D), W2)
    s2a, s2b = _agg_call(m2, src, dst)
    h3 = _tcC_call(h2, m2, s2a, s2b, d0, d1, b2.reshape(1, D))
    return h3
